# baseline (speedup 1.0000x reference)
import functools

import jax
import jax.numpy as jnp
from jax import lax
from jax.experimental import pallas as pl
from jax.experimental.pallas import tpu as pltpu
from jax.experimental.pallas import tpu_sc as plsc

B = 16384
D = 32
NUM_TABLES = 26
NUM = 13
NUM_PAD = 16
H = 256
XW = NUM_PAD + NUM_TABLES * D
ORDER_A = list(range(2, 14))
ORDER_B = list(range(14, NUM_TABLES))
ORDER_C = [0, 1]
NTA = len(ORDER_A)
WA = NUM_PAD + NTA * D
WB = len(ORDER_B) * D
WC = len(ORDER_C) * D

_info = plsc.get_sparse_core_info()
NC = _info.num_cores
NS = _info.num_subcores
NW = NC * NS
BPW = B // NW


def _make_body(n_tables, with_num):
    def body(num_hbm, idx_hbm, *rest):
        tables = rest[:n_tables]
        x_out = rest[n_tables]
        idx_all, rows, sem_g, sem_w0, sem_w1 = rest[n_tables + 1:]

        wid = lax.axis_index("s") * NC + lax.axis_index("c")
        base = wid * BPW
        col0 = NUM_PAD if with_num else 0

        pltpu.sync_copy(idx_hbm.at[wid], idx_all)
        if with_num:
            pltpu.sync_copy(num_hbm.at[pl.ds(base, BPW)],
                            x_out.at[pl.ds(base, BPW), pl.ds(0, NUM_PAD)])

        sem_w = (sem_w0, sem_w1)
        pending = [None, None]
        for i in range(n_tables):
            p = i % 2
            if pending[p] is not None:
                pending[p].wait()
            g = pltpu.async_copy(tables[i].at[idx_all.at[i]],
                                 rows.at[p], sem_g)
            g.wait()
            pending[p] = pltpu.async_copy(
                rows.at[p],
                x_out.at[pl.ds(base, BPW), pl.ds(col0 + D * i, D)],
                sem_w[p],
            )
        pending[0].wait()
        pending[1].wait()

    return body


def _make_sc(n_tables, with_num, width):
    return functools.partial(
        pl.kernel,
        mesh=plsc.VectorSubcoreMesh(core_axis_name="c", subcore_axis_name="s"),
        out_type=jax.ShapeDtypeStruct((B, width), jnp.float32),
        compiler_params=pltpu.CompilerParams(use_tc_tiling_on_sc=False),
        scratch_types=[
            pltpu.VMEM((n_tables, BPW), jnp.int32),
            pltpu.VMEM((2, BPW, D), jnp.float32),
            pltpu.SemaphoreType.DMA,
            pltpu.SemaphoreType.DMA,
            pltpu.SemaphoreType.DMA,
        ],
    )(_make_body(n_tables, with_num))


_sc_a = _make_sc(NTA, True, WA)
_sc_b = _make_sc(len(ORDER_B), False, WB)
_sc_c = _make_sc(len(ORDER_C), False, WC)


TB = 1024


def _mm_body(xa_ref, xb_ref, xc_ref, w_ref, b_ref, o_ref):
    x = jnp.concatenate([xa_ref[...], xb_ref[...], xc_ref[...]], axis=1)
    o_ref[...] = (
        jnp.dot(x, w_ref[...], preferred_element_type=jnp.float32)
        + b_ref[...]
    )


_tc_matmul = pl.pallas_call(
    _mm_body,
    grid=(B // TB,),
    in_specs=[
        pl.BlockSpec((TB, WA), lambda i: (i, 0)),
        pl.BlockSpec((TB, WB), lambda i: (i, 0)),
        pl.BlockSpec((TB, WC), lambda i: (i, 0)),
        pl.BlockSpec((XW, H), lambda i: (0, 0)),
        pl.BlockSpec((1, H), lambda i: (0, 0)),
    ],
    out_specs=pl.BlockSpec((TB, H), lambda i: (i, 0)),
    out_shape=jax.ShapeDtypeStruct((B, H), jnp.float32),
)


def kernel(numerical, cat_0, cat_1, cat_2, cat_3, cat_4, cat_5, cat_6, cat_7, cat_8, cat_9, cat_10, cat_11, cat_12, cat_13, cat_14, cat_15, cat_16, cat_17, cat_18, cat_19, cat_20, cat_21, cat_22, cat_23, cat_24, cat_25, emb_0, emb_1, emb_2, emb_3, emb_4, emb_5, emb_6, emb_7, emb_8, emb_9, emb_10, emb_11, emb_12, emb_13, emb_14, emb_15, emb_16, emb_17, emb_18, emb_19, emb_20, emb_21, emb_22, emb_23, emb_24, emb_25, W, b):
    embs = [emb_0, emb_1, emb_2, emb_3, emb_4, emb_5, emb_6, emb_7, emb_8,
            emb_9, emb_10, emb_11, emb_12, emb_13, emb_14, emb_15, emb_16,
            emb_17, emb_18, emb_19, emb_20, emb_21, emb_22, emb_23, emb_24,
            emb_25]
    cats = jnp.stack(
        [cat_0, cat_1, cat_2, cat_3, cat_4, cat_5, cat_6, cat_7, cat_8,
         cat_9, cat_10, cat_11, cat_12, cat_13, cat_14, cat_15, cat_16,
         cat_17, cat_18, cat_19, cat_20, cat_21, cat_22, cat_23, cat_24,
         cat_25], axis=0).astype(jnp.int32)
    idx = cats.reshape(NUM_TABLES, NW, BPW).transpose(1, 0, 2)
    num_pad = jnp.pad(numerical, ((0, 0), (0, NUM_PAD - NUM)))
    xa = _sc_a(num_pad, idx[:, ORDER_A], *[embs[i] for i in ORDER_A])
    xb = _sc_b(num_pad, idx[:, ORDER_B], *[embs[i] for i in ORDER_B])
    xc = _sc_c(num_pad, idx[:, ORDER_C], *[embs[i] for i in ORDER_C])
    W_pad = jnp.concatenate(
        [W[:NUM], jnp.zeros((NUM_PAD - NUM, H), W.dtype)]
        + [W[NUM + D * i: NUM + D * (i + 1)]
           for i in ORDER_A + ORDER_B + ORDER_C],
        axis=0)
    return _tc_matmul(xa, xb, xc, W_pad, b.reshape(1, H))


# speedup vs baseline: 1.0037x; 1.0037x over previous
import functools

import jax
import jax.numpy as jnp
from jax import lax
from jax.experimental import pallas as pl
from jax.experimental.pallas import tpu as pltpu
from jax.experimental.pallas import tpu_sc as plsc

B = 16384
D = 32
NUM_TABLES = 26
NUM = 13
NUM_PAD = 16
H = 256
XW = NUM_PAD + NUM_TABLES * D
ORDER_A = list(range(2, NUM_TABLES))
ORDER_B = [0, 1]
NTA = len(ORDER_A)
WA = NUM_PAD + NTA * D
WB = len(ORDER_B) * D

_info = plsc.get_sparse_core_info()
NC = _info.num_cores
NS = _info.num_subcores
NW = NC * NS
BPW = B // NW


def _make_body(n_tables, with_num):
    def body(num_hbm, idx_hbm, *rest):
        tables = rest[:n_tables]
        x_out = rest[n_tables]
        idx_all, rows, sem_g, sem_w0, sem_w1 = rest[n_tables + 1:]

        wid = lax.axis_index("s") * NC + lax.axis_index("c")
        base = wid * BPW
        col0 = NUM_PAD if with_num else 0

        pltpu.sync_copy(idx_hbm.at[wid], idx_all)
        if with_num:
            pltpu.sync_copy(num_hbm.at[pl.ds(base, BPW)],
                            x_out.at[pl.ds(base, BPW), pl.ds(0, NUM_PAD)])

        sem_w = (sem_w0, sem_w1)
        pending = [None, None]
        for i in range(n_tables):
            p = i % 2
            if pending[p] is not None:
                pending[p].wait()
            g = pltpu.async_copy(tables[i].at[idx_all.at[i]],
                                 rows.at[p], sem_g)
            g.wait()
            pending[p] = pltpu.async_copy(
                rows.at[p],
                x_out.at[pl.ds(base, BPW), pl.ds(col0 + D * i, D)],
                sem_w[p],
            )
        pending[0].wait()
        pending[1].wait()

    return body


def _make_sc(n_tables, with_num, width):
    return functools.partial(
        pl.kernel,
        mesh=plsc.VectorSubcoreMesh(core_axis_name="c", subcore_axis_name="s"),
        out_type=jax.ShapeDtypeStruct((B, width), jnp.float32),
        compiler_params=pltpu.CompilerParams(use_tc_tiling_on_sc=False),
        scratch_types=[
            pltpu.VMEM((n_tables, BPW), jnp.int32),
            pltpu.VMEM((2, BPW, D), jnp.float32),
            pltpu.SemaphoreType.DMA,
            pltpu.SemaphoreType.DMA,
            pltpu.SemaphoreType.DMA,
        ],
    )(_make_body(n_tables, with_num))


_sc_a = _make_sc(NTA, True, WA)
_sc_b = _make_sc(len(ORDER_B), False, WB)


TB = 1024


def _mm_body(xa_ref, xb_ref, w_ref, b_ref, o_ref):
    x = jnp.concatenate([xa_ref[...], xb_ref[...]], axis=1)
    o_ref[...] = (
        jnp.dot(x, w_ref[...], preferred_element_type=jnp.float32)
        + b_ref[...]
    )


_tc_matmul = pl.pallas_call(
    _mm_body,
    grid=(B // TB,),
    in_specs=[
        pl.BlockSpec((TB, WA), lambda i: (i, 0)),
        pl.BlockSpec((TB, WB), lambda i: (i, 0)),
        pl.BlockSpec((XW, H), lambda i: (0, 0)),
        pl.BlockSpec((1, H), lambda i: (0, 0)),
    ],
    out_specs=pl.BlockSpec((TB, H), lambda i: (i, 0)),
    out_shape=jax.ShapeDtypeStruct((B, H), jnp.float32),
)


def kernel(numerical, cat_0, cat_1, cat_2, cat_3, cat_4, cat_5, cat_6, cat_7, cat_8, cat_9, cat_10, cat_11, cat_12, cat_13, cat_14, cat_15, cat_16, cat_17, cat_18, cat_19, cat_20, cat_21, cat_22, cat_23, cat_24, cat_25, emb_0, emb_1, emb_2, emb_3, emb_4, emb_5, emb_6, emb_7, emb_8, emb_9, emb_10, emb_11, emb_12, emb_13, emb_14, emb_15, emb_16, emb_17, emb_18, emb_19, emb_20, emb_21, emb_22, emb_23, emb_24, emb_25, W, b):
    embs = [emb_0, emb_1, emb_2, emb_3, emb_4, emb_5, emb_6, emb_7, emb_8,
            emb_9, emb_10, emb_11, emb_12, emb_13, emb_14, emb_15, emb_16,
            emb_17, emb_18, emb_19, emb_20, emb_21, emb_22, emb_23, emb_24,
            emb_25]
    cats = jnp.stack(
        [cat_0, cat_1, cat_2, cat_3, cat_4, cat_5, cat_6, cat_7, cat_8,
         cat_9, cat_10, cat_11, cat_12, cat_13, cat_14, cat_15, cat_16,
         cat_17, cat_18, cat_19, cat_20, cat_21, cat_22, cat_23, cat_24,
         cat_25], axis=0).astype(jnp.int32)
    idx = cats.reshape(NUM_TABLES, NW, BPW).transpose(1, 0, 2)
    num_pad = jnp.pad(numerical, ((0, 0), (0, NUM_PAD - NUM)))
    xa = _sc_a(num_pad, idx[:, ORDER_A], *[embs[i] for i in ORDER_A])
    xb = _sc_b(num_pad, idx[:, ORDER_B], *[embs[i] for i in ORDER_B])
    W_pad = jnp.concatenate(
        [W[:NUM], jnp.zeros((NUM_PAD - NUM, H), W.dtype)]
        + [W[NUM + D * i: NUM + D * (i + 1)] for i in ORDER_A + ORDER_B],
        axis=0)
    return _tc_matmul(xa, xb, W_pad, b.reshape(1, H))
